# Initial kernel scaffold; baseline (speedup 1.0000x reference)
import jax
import jax.numpy as jnp
from jax.experimental import pallas as pl


def kernel(x, edge_index, norm):
    raise NotImplementedError("write your pallas kernel here")


# speedup vs baseline: 4.9750x; 4.9750x over previous
import functools

import jax
import jax.numpy as jnp
from jax import lax
from jax.experimental import pallas as pl
from jax.experimental.pallas import tpu as pltpu
from jax.experimental.pallas import tpu_sc as plsc

N_N = 10000
N_E = 320000
D = 128
NC, NS = 2, 16
NW = NC * NS
EPW = N_E // NW
CHUNK = 80
NCH = EPW // CHUNK
RPS = 624
ZB = 48
DB = 208
REM = N_N - NS * RPS

_mesh = plsc.VectorSubcoreMesh(
    core_axis_name="c", subcore_axis_name="s", num_cores=NC, num_subcores=NS
)


@functools.partial(
    pl.kernel,
    out_type=jax.ShapeDtypeStruct((NC, N_N, D), jnp.float32),
    mesh=_mesh,
    scratch_types=[
        pltpu.VMEM((CHUNK,), jnp.int32),
        pltpu.VMEM((NCH, CHUNK), jnp.int32),
        pltpu.VMEM((CHUNK,), jnp.float32),
        pltpu.VMEM((CHUNK, D), jnp.float32),
        pltpu.VMEM_SHARED((N_N, D), jnp.float32),
    ],
)
def _sc_gather_scatter(src_hbm, dst_hbm, norm_hbm, x_hbm, out_hbm,
                       src_v, dst_v, norm_v, msg_v, h_sh):
    c = lax.axis_index("c")
    s = lax.axis_index("s")
    wid = s * NC + c
    r0 = pl.multiple_of(s * RPS, 8)

    pltpu.sync_copy(dst_hbm.at[wid], dst_v)

    zero = jnp.zeros((16,), jnp.float32)

    def _zrow(i, _):
        for j in range(D // 16):
            msg_v[i, pl.ds(16 * j, 16)] = zero
        return 0

    lax.fori_loop(0, ZB, _zrow, 0)
    for t in range(RPS // ZB):
        pltpu.sync_copy(msg_v.at[pl.ds(0, ZB), :],
                        h_sh.at[pl.ds(r0 + t * ZB, ZB), :])

    @pl.when(s == NS - 1)
    def _zero_tail():
        pltpu.sync_copy(msg_v.at[pl.ds(0, REM), :],
                        h_sh.at[pl.ds(NS * RPS, REM), :])

    plsc.subcore_barrier()

    e0 = wid * EPW

    def _chunk(k, _):
        pltpu.sync_copy(src_hbm.at[pl.ds(e0 + k * CHUNK, CHUNK)], src_v)
        pltpu.sync_copy(norm_hbm.at[pl.ds(e0 + k * CHUNK, CHUNK)], norm_v)
        pltpu.sync_copy(x_hbm.at[src_v], msg_v)

        def _scale(g, _):
            nv16 = norm_v[pl.ds(g * 16, 16)]
            for i in range(16):
                nvec = lax.gather(
                    nv16, jnp.full((16, 1), i, jnp.int32),
                    lax.GatherDimensionNumbers(
                        offset_dims=(), collapsed_slice_dims=(0,),
                        start_index_map=(0,)),
                    slice_sizes=(1,),
                    mode=lax.GatherScatterMode.PROMISE_IN_BOUNDS)
                e = g * 16 + i
                for j in range(D // 16):
                    msg_v[e, pl.ds(16 * j, 16)] = (
                        msg_v[e, pl.ds(16 * j, 16)] * nvec)
            return 0

        lax.fori_loop(0, CHUNK // 16, _scale, 0)
        pltpu.sync_copy(msg_v, h_sh.at[dst_v.at[k]], add=True)
        return 0

    lax.fori_loop(0, NCH, _chunk, 0)
    plsc.subcore_barrier()

    for t in range(RPS // DB):
        pltpu.sync_copy(h_sh.at[pl.ds(r0 + t * DB, DB), :],
                        out_hbm.at[c, pl.ds(r0 + t * DB, DB), :])

    @pl.when(s == NS - 1)
    def _drain_tail():
        pltpu.sync_copy(h_sh.at[pl.ds(NS * RPS, REM), :],
                        out_hbm.at[c, pl.ds(NS * RPS, REM), :])


def _finalize_body(p_ref, o_ref):
    a = p_ref[0] + p_ref[1]
    l2 = jnp.sqrt(jnp.sum(a * a, axis=1, keepdims=True))
    o_ref[...] = a / jnp.maximum(l2, 1e-12)


_R = 1000


def _finalize(partials):
    return pl.pallas_call(
        _finalize_body,
        grid=(N_N // _R,),
        in_specs=[pl.BlockSpec((NC, _R, D), lambda i: (0, i, 0))],
        out_specs=pl.BlockSpec((_R, D), lambda i: (i, 0)),
        out_shape=jax.ShapeDtypeStruct((N_N, D), jnp.float32),
    )(partials)


@jax.jit
def kernel(x, edge_index, norm):
    src = edge_index[0]
    dst = edge_index[1].reshape(NW, NCH, CHUNK)
    nrm = norm.reshape(N_E)
    partials = _sc_gather_scatter(src, dst, nrm, x)
    return _finalize(partials)
